# Initial kernel scaffold; baseline (speedup 1.0000x reference)
import jax
import jax.numpy as jnp
from jax.experimental import pallas as pl


def kernel(laplacian_indices, laplacian_values, features, W, b):
    raise NotImplementedError("write your pallas kernel here")


# speedup vs baseline: 4.4518x; 4.4518x over previous
import functools

import jax
import jax.numpy as jnp
from jax import lax
from jax.experimental import pallas as pl
from jax.experimental.pallas import tpu as pltpu
from jax.experimental.pallas import tpu_sc as plsc

N = 10000
E = 320000
D = 128

NC = 2
NS = 16
NW = NC * NS
EPW = E // NW
C = 80
NCHUNK = EPW // C
RPT = 640
RPT_LAST_CHUNKS = (N - 15 * RPT) // C
RPT_CHUNKS = RPT // C


def _mm_body(f_ref, wt_ref, b_ref, o_ref):
    o_ref[...] = (
        jnp.dot(f_ref[...], wt_ref[...], preferred_element_type=jnp.float32)
        + b_ref[...]
    )


def _add_body(p_ref, o_ref):
    o_ref[...] = p_ref[0] + p_ref[1]


def _sc_spmm_body(support_hbm, rows_hbm, cols_hbm, vals_hbm, out_hbm,
                  colbuf, rowbuf, valbuf, rowsbuf, acc, sem):
    cid = lax.axis_index("c")
    sid = lax.axis_index("s")
    wid = cid * NS + sid
    zero16 = jnp.zeros((16,), jnp.float32)

    def zrow(g, carry):
        for j in range(D // 16):
            rowsbuf[g, pl.ds(j * 16, 16)] = zero16
        return carry
    lax.fori_loop(0, C, zrow, 0)

    row0 = sid * RPT
    nch = jnp.where(sid == NS - 1, RPT_LAST_CHUNKS, RPT_CHUNKS)

    def zacc(k, carry):
        pltpu.sync_copy(rowsbuf, acc.at[pl.ds(row0 + k * C, C)])
        return carry
    lax.fori_loop(0, nch, zacc, 0)

    plsc.subcore_barrier()

    base_w = wid * EPW

    def chunk(ci, carry):
        base = base_w + ci * C
        pltpu.sync_copy(cols_hbm.at[pl.ds(base, C)], colbuf)
        pltpu.sync_copy(rows_hbm.at[pl.ds(base, C)], rowbuf)
        pltpu.sync_copy(vals_hbm.at[pl.ds(base, C)], valbuf)
        pltpu.async_copy(support_hbm.at[colbuf], rowsbuf, sem).wait()

        def scale(g, inner):
            vv = valbuf[pl.ds(g * 16, 16)]
            for i2 in range(16):
                r = g * 16 + i2
                s = lax.gather(
                    vv, jnp.full((16, 1), i2, jnp.int32),
                    lax.GatherDimensionNumbers(
                        offset_dims=(), collapsed_slice_dims=(0,),
                        start_index_map=(0,)),
                    (1,), mode=lax.GatherScatterMode.PROMISE_IN_BOUNDS)
                for j in range(D // 16):
                    rowsbuf[r, pl.ds(j * 16, 16)] = (
                        rowsbuf[r, pl.ds(j * 16, 16)] * s)
            return inner
        lax.fori_loop(0, C // 16, scale, 0)

        pltpu.sync_copy(rowsbuf, acc.at[rowbuf], add=True)
        return carry
    lax.fori_loop(0, NCHUNK, chunk, 0)

    plsc.subcore_barrier()

    def wb(k, carry):
        sl = pl.ds(row0 + k * C, C)
        pltpu.sync_copy(acc.at[sl], rowsbuf)
        pltpu.sync_copy(rowsbuf, out_hbm.at[cid, sl])
        return carry
    lax.fori_loop(0, nch, wb, 0)


_sc_spmm = functools.partial(
    pl.kernel,
    out_type=jax.ShapeDtypeStruct((NC, N, D), jnp.float32),
    mesh=plsc.VectorSubcoreMesh(
        core_axis_name="c", subcore_axis_name="s",
        num_cores=NC, num_subcores=NS),
    scratch_types=[
        pltpu.VMEM((C,), jnp.int32),
        pltpu.VMEM((C,), jnp.int32),
        pltpu.VMEM((C,), jnp.float32),
        pltpu.VMEM((C, D), jnp.float32),
        pltpu.VMEM_SHARED((N, D), jnp.float32),
        pltpu.SemaphoreType.DMA,
    ],
)(_sc_spmm_body)


def kernel(laplacian_indices, laplacian_values, features, W, b):
    rows = laplacian_indices[0]
    cols = laplacian_indices[1]
    wt = W.T
    b2 = b.reshape(1, D)

    support = pl.pallas_call(
        _mm_body,
        grid=(10,),
        in_specs=[
            pl.BlockSpec((N // 10, D), lambda i: (i, 0)),
            pl.BlockSpec((D, D), lambda i: (0, 0)),
            pl.BlockSpec((1, D), lambda i: (0, 0)),
        ],
        out_specs=pl.BlockSpec((N // 10, D), lambda i: (i, 0)),
        out_shape=jax.ShapeDtypeStruct((N, D), jnp.float32),
    )(features, wt, b2)

    partials = _sc_spmm(support, rows, cols, laplacian_values)

    out = pl.pallas_call(
        _add_body,
        grid=(10,),
        in_specs=[pl.BlockSpec((NC, N // 10, D), lambda i: (0, i, 0))],
        out_specs=pl.BlockSpec((N // 10, D), lambda i: (i, 0)),
        out_shape=jax.ShapeDtypeStruct((N, D), jnp.float32),
    )(partials)
    return out
